# baseline (speedup 1.0000x reference)
import functools

import jax
import jax.numpy as jnp
import numpy as np
from jax import lax
from jax.experimental import pallas as pl
from jax.experimental.pallas import tpu as pltpu
from jax.experimental.pallas import tpu_sc as plsc

_VOCAB = 1000
_EMBED = 64
_N_OUT = 10
_BATCH = 4096
_SEQ = 200

_NC = 2
_NS = 16
_NW = _NC * _NS
_ROWS = _BATCH // _NW
_L = 16
_WPR = _EMBED // 2
_PCH = _WPR // _L
_NCHUNK = (_SEQ + _L - 1) // _L
_VPAD = 1024
_TPT = _VPAD // _NS
_PUNROLL = 4
_OSTRIDE = 128


def _sc_pool(idx_t, table_pad):
    mesh = plsc.VectorSubcoreMesh(
        core_axis_name="c", subcore_axis_name="s",
        num_cores=_NC, num_subcores=_NS)

    @functools.partial(
        pl.kernel,
        out_type=jax.ShapeDtypeStruct((_BATCH * _OSTRIDE,), jnp.int32),
        mesh=mesh,
        scratch_types=[
            pltpu.VMEM((_TPT, _EMBED), jnp.float32),
            pltpu.VMEM((_VPAD * _WPR,), jnp.int32),
            pltpu.VMEM((_SEQ, _ROWS), jnp.int32),
            pltpu.VMEM((_ROWS * _OSTRIDE,), jnp.int32),
            pltpu.VMEM_SHARED((_VPAD * _WPR,), jnp.int32),
            pltpu.SemaphoreType.DMA,
        ],
        compiler_params=pltpu.CompilerParams(needs_layout_passes=False),
    )
    def k(idx_hbm, table_hbm, out_hbm, tf32_v, table_v, idx_v,
          out_v, shared_v, idx_sem):
        sid = lax.axis_index("s")
        wid = sid * _NC + lax.axis_index("c")
        idx_cpy = pltpu.async_copy(
            idx_hbm.at[:, pl.ds(wid * _ROWS, _ROWS)], idx_v, idx_sem)

        pltpu.sync_copy(table_hbm.at[pl.ds(sid * _TPT, _TPT)], tf32_v)

        def pack_body(i, carry):
            for dr in range(_PUNROLL):
                r = i * _PUNROLL + dr
                for h2 in range(_PCH):
                    a = tf32_v[r, pl.ds(h2 * 32, _L)]
                    b = tf32_v[r, pl.ds(h2 * 32 + _L, _L)]
                    w = plsc.pack(a, b, format=plsc.PackFormat.INTERLEAVED)
                    out_v[pl.ds(r * _WPR + h2 * _L, _L)] = plsc.bitcast(
                        w, jnp.int32)
            return carry

        lax.fori_loop(0, _TPT // _PUNROLL, pack_body, 0)
        pltpu.sync_copy(out_v.at[pl.ds(0, _TPT * _WPR)],
                        shared_v.at[pl.ds(sid * (_TPT * _WPR), _TPT * _WPR)])
        plsc.subcore_barrier()
        pltpu.sync_copy(shared_v, table_v)
        idx_cpy.wait()

        lane_off = [lax.iota(jnp.int32, _L) + (ci * _L) for ci in range(_PCH)]
        tok_iota = lax.iota(jnp.int32, _L)
        neg = jnp.full((2 * _L,), -jnp.inf, jnp.bfloat16)

        def row_body(r2, carry):
            rvecs = [jnp.full((_L,), 2 * r2 + j, jnp.int32) for j in range(2)]

            def chunk_body(c, accs):
                start = jnp.minimum(c * _L, _SEQ - _L)
                bases = [plsc.load_gather(idx_v, [tok_iota + start, rv]) * _WPR
                         for rv in rvecs]
                accs = list(accs)
                for t in range(_L):
                    tsel = jnp.full((_L,), t, jnp.int32)
                    for j in range(2):
                        tb = bases[j].at[tsel].get(mode="promise_in_bounds")
                        for ci in range(_PCH):
                            accs[j * _PCH + ci] = jnp.maximum(
                                accs[j * _PCH + ci],
                                plsc.bitcast(
                                    plsc.load_gather(table_v,
                                                     [tb + lane_off[ci]]),
                                    jnp.bfloat16))
                return tuple(accs)

            accs = lax.fori_loop(0, _NCHUNK, chunk_body, (neg,) * (2 * _PCH))
            for j in range(2):
                for ci in range(_PCH):
                    out_v[pl.ds((2 * r2 + j) * _OSTRIDE + ci * _L,
                                _L)] = plsc.bitcast(accs[j * _PCH + ci],
                                                    jnp.int32)
            return carry

        lax.fori_loop(0, _ROWS // 2, row_body, 0)
        pltpu.sync_copy(out_v,
                        out_hbm.at[pl.ds(wid * (_ROWS * _OSTRIDE),
                                         _ROWS * _OSTRIDE)])

    return k(idx_t, table_pad)


_PAD_OUT = 128
_PERM_LO = np.array([h2 * 32 + k for h2 in range(_PCH) for k in range(_L)])
_PERM_HI = _PERM_LO + _L


def _tc_head(pooled_packed, w_lo, w_hi, b_pad):
    blk = 512

    def body(p_ref, wl_ref, wh_ref, b_ref, o_ref):
        x = p_ref[...][:, :_WPR]
        lo = lax.bitcast_convert_type(x << 16, jnp.float32)
        hi = lax.bitcast_convert_type(
            x & jnp.int32(-65536), jnp.float32)
        dn = (((0,), (1,)), ((), ()))
        logits = (lax.dot_general(wl_ref[...], lo, dn,
                                  preferred_element_type=jnp.float32)
                  + lax.dot_general(wh_ref[...], hi, dn,
                                    preferred_element_type=jnp.float32)
                  + b_ref[...])
        m = jnp.max(logits, axis=0, keepdims=True)
        e = jnp.exp(logits - m)
        o_ref[...] = e / jnp.sum(e, axis=0, keepdims=True)

    return pl.pallas_call(
        body,
        grid=(_BATCH // blk,),
        in_specs=[
            pl.BlockSpec((blk, _OSTRIDE), lambda i: (i, 0)),
            pl.BlockSpec((_WPR, _PAD_OUT), lambda i: (0, 0)),
            pl.BlockSpec((_WPR, _PAD_OUT), lambda i: (0, 0)),
            pl.BlockSpec((_PAD_OUT, 1), lambda i: (0, 0)),
        ],
        out_specs=pl.BlockSpec((_PAD_OUT, blk), lambda i: (0, i)),
        out_shape=jax.ShapeDtypeStruct((_PAD_OUT, _BATCH), jnp.float32),
    )(pooled_packed, w_lo, w_hi, b_pad)


def kernel(inputs, table, W, b):
    table_pad = jnp.pad(table, ((0, _VPAD - _VOCAB), (0, 0)))
    pooled_packed = _sc_pool(inputs.T, table_pad).reshape(_BATCH, _OSTRIDE)
    pad = ((0, 0), (0, _PAD_OUT - _N_OUT))
    w_lo = jnp.pad(W[_PERM_LO, :], pad)
    w_hi = jnp.pad(W[_PERM_HI, :], pad)
    b_pad = jnp.pad(b, (0, _PAD_OUT - _N_OUT),
                    constant_values=-1e30).reshape(_PAD_OUT, 1)
    out_t = _tc_head(pooled_packed, w_lo, w_hi, b_pad)
    return out_t[:_N_OUT, :].T


# speedup vs baseline: 1.1209x; 1.1209x over previous
import functools

import jax
import jax.numpy as jnp
import numpy as np
from jax import lax
from jax.experimental import pallas as pl
from jax.experimental.pallas import tpu as pltpu
from jax.experimental.pallas import tpu_sc as plsc

_VOCAB = 1000
_EMBED = 64
_N_OUT = 10
_BATCH = 4096
_SEQ = 200

_NC = 2
_NS = 16
_NW = _NC * _NS
_ROWS = _BATCH // _NW
_L = 16
_WPR = _EMBED // 2
_PCH = _WPR // _L
_NCHUNK = (_SEQ + _L - 1) // _L
_VPAD = 1024
_TPT = _VPAD // _NS
_PUNROLL = 4
_OSTRIDE = 128


def _sc_pool(idx_t, table_pad):
    mesh = plsc.VectorSubcoreMesh(
        core_axis_name="c", subcore_axis_name="s",
        num_cores=_NC, num_subcores=_NS)

    @functools.partial(
        pl.kernel,
        out_type=jax.ShapeDtypeStruct((_BATCH * _OSTRIDE,), jnp.int32),
        mesh=mesh,
        scratch_types=[
            pltpu.VMEM((_TPT, _EMBED), jnp.float32),
            pltpu.VMEM((_VPAD * _WPR,), jnp.int32),
            pltpu.VMEM((_SEQ, _ROWS), jnp.int32),
            pltpu.VMEM((_ROWS * _OSTRIDE,), jnp.int32),
            pltpu.VMEM_SHARED((_VPAD * _WPR,), jnp.int32),
            pltpu.SemaphoreType.DMA,
        ],
        compiler_params=pltpu.CompilerParams(needs_layout_passes=False),
    )
    def k(idx_hbm, table_hbm, out_hbm, tf32_v, table_v, idx_v,
          out_v, shared_v, idx_sem):
        sid = lax.axis_index("s")
        wid = sid * _NC + lax.axis_index("c")
        idx_cpy = pltpu.async_copy(
            idx_hbm.at[:, pl.ds(wid * _ROWS, _ROWS)], idx_v, idx_sem)

        pltpu.sync_copy(table_hbm.at[pl.ds(sid * _TPT, _TPT)], tf32_v)

        def pack_body(i, carry):
            for dr in range(_PUNROLL):
                r = i * _PUNROLL + dr
                for h2 in range(_PCH):
                    a = tf32_v[r, pl.ds(h2 * 32, _L)]
                    b = tf32_v[r, pl.ds(h2 * 32 + _L, _L)]
                    w = plsc.pack(a, b, format=plsc.PackFormat.INTERLEAVED)
                    out_v[pl.ds(r * _WPR + h2 * _L, _L)] = plsc.bitcast(
                        w, jnp.int32)
            return carry

        lax.fori_loop(0, _TPT // _PUNROLL, pack_body, 0)
        pltpu.sync_copy(out_v.at[pl.ds(0, _TPT * _WPR)],
                        shared_v.at[pl.ds(sid * (_TPT * _WPR), _TPT * _WPR)])
        plsc.subcore_barrier()
        pltpu.sync_copy(shared_v, table_v)
        idx_cpy.wait()

        lane_off = [lax.iota(jnp.int32, _L) + (ci * _L) for ci in range(_PCH)]
        tok_iota = lax.iota(jnp.int32, _L)
        neg = jnp.full((2 * _L,), -jnp.inf, jnp.bfloat16)

        def row_body(r, carry):
            rvec = jnp.full((_L,), r, jnp.int32)

            def chunk_body(c, accs):
                start = jnp.minimum(c * _L, _SEQ - _L)
                base = plsc.load_gather(idx_v, [tok_iota + start, rvec]) * _WPR
                for t in range(_L):
                    tb = base.at[jnp.full((_L,), t, jnp.int32)].get(
                        mode="promise_in_bounds")
                    accs = tuple(
                        jnp.maximum(a, plsc.bitcast(
                            plsc.load_gather(table_v, [tb + lane_off[ci]]),
                            jnp.bfloat16))
                        for ci, a in enumerate(accs))
                return accs

            accs = lax.fori_loop(0, _NCHUNK, chunk_body, (neg,) * _PCH)
            for ci in range(_PCH):
                out_v[pl.ds(r * _OSTRIDE + ci * _L, _L)] = plsc.bitcast(
                    accs[ci], jnp.int32)
            return carry

        lax.fori_loop(0, _ROWS, row_body, 0)
        pltpu.sync_copy(out_v,
                        out_hbm.at[pl.ds(wid * (_ROWS * _OSTRIDE),
                                         _ROWS * _OSTRIDE)])

    return k(idx_t, table_pad)


_PAD_OUT = 128
_PERM_LO = np.array([h2 * 32 + k for h2 in range(_PCH) for k in range(_L)])
_PERM_HI = _PERM_LO + _L


def _tc_head(pooled_packed, w_lo, w_hi, b_pad):
    blk = 512

    def body(p_ref, wl_ref, wh_ref, b_ref, o_ref):
        x = p_ref[...][:, :_WPR]
        lo = lax.bitcast_convert_type(x << 16, jnp.float32)
        hi = lax.bitcast_convert_type(
            x & jnp.int32(-65536), jnp.float32)
        dn = (((0,), (1,)), ((), ()))
        logits = (lax.dot_general(wl_ref[...], lo, dn,
                                  preferred_element_type=jnp.float32)
                  + lax.dot_general(wh_ref[...], hi, dn,
                                    preferred_element_type=jnp.float32)
                  + b_ref[...])
        m = jnp.max(logits, axis=0, keepdims=True)
        e = jnp.exp(logits - m)
        o_ref[...] = e / jnp.sum(e, axis=0, keepdims=True)

    return pl.pallas_call(
        body,
        grid=(_BATCH // blk,),
        in_specs=[
            pl.BlockSpec((blk, _OSTRIDE), lambda i: (i, 0)),
            pl.BlockSpec((_WPR, _PAD_OUT), lambda i: (0, 0)),
            pl.BlockSpec((_WPR, _PAD_OUT), lambda i: (0, 0)),
            pl.BlockSpec((_PAD_OUT, 1), lambda i: (0, 0)),
        ],
        out_specs=pl.BlockSpec((_PAD_OUT, blk), lambda i: (0, i)),
        out_shape=jax.ShapeDtypeStruct((_PAD_OUT, _BATCH), jnp.float32),
    )(pooled_packed, w_lo, w_hi, b_pad)


def kernel(inputs, table, W, b):
    table_pad = jnp.pad(table, ((0, _VPAD - _VOCAB), (0, 0)))
    pooled_packed = _sc_pool(inputs.T, table_pad).reshape(_BATCH, _OSTRIDE)
    pad = ((0, 0), (0, _PAD_OUT - _N_OUT))
    w_lo = jnp.pad(W[_PERM_LO, :], pad)
    w_hi = jnp.pad(W[_PERM_HI, :], pad)
    b_pad = jnp.pad(b, (0, _PAD_OUT - _N_OUT),
                    constant_values=-1e30).reshape(_PAD_OUT, 1)
    out_t = _tc_head(pooled_packed, w_lo, w_hi, b_pad)
    return out_t[:_N_OUT, :].T
